# baseline (speedup 1.0000x reference)
import functools

import jax
import jax.numpy as jnp
from jax.experimental import pallas as pl
from jax.experimental.pallas import tpu as pltpu

N = 64
E = 1024
H = 32
EPS = 20.0
EB = 128
CB = 128
NS = 16
F32 = jnp.float32


def _dot(a, b):
    return jnp.dot(a, b, preferred_element_type=F32)


def _bdot(a, b):
    return jnp.dot(a.astype(jnp.bfloat16), b.astype(jnp.bfloat16),
                   preferred_element_type=F32)


def _msgs_body(bij_ref, bi_ref, mi_ref, mj_ref, os_ref, od_ref, ds_ref,
               dd_ref, cmi_ref, cmj_ref, p1_ref, p2_ref, ef_ref):
    bijb = bij_ref[...]
    bi2 = bi_ref[...]
    ntoe1 = _dot(os_ref[...], bi2)
    ntoe2 = _dot(od_ref[...], bi2)
    t = EPS * (bijb + ntoe2[:, None, :])
    m1 = jnp.max(t, axis=2)
    cmi = (m1 + jnp.log(jnp.sum(jnp.exp(t - m1[:, :, None]), axis=2))) / EPS
    t2 = EPS * (bijb + ntoe1[:, :, None])
    m2 = jnp.max(t2, axis=1)
    cmj = (m2 + jnp.log(jnp.sum(jnp.exp(t2 - m2[:, None, :]), axis=1))) / EPS
    p1 = jnp.sum(bijb * ds_ref[...][:, :, None], axis=1)
    p2 = jnp.sum(bijb * dd_ref[...][:, None, :], axis=2)
    cmi_ref[...] = cmi
    cmj_ref[...] = cmj
    p1_ref[...] = p1
    p2_ref[...] = p2
    zero = jnp.zeros_like(cmi)
    ef_ref[...] = jnp.stack(
        [mi_ref[...], mj_ref[...], cmi, cmj, ntoe1, ntoe2] + [zero] * 2,
        axis=-1)


def _msgs_call(bij3, bi2, msgi2, msgj2, osrc, odst, ds, dd):
    eb = pl.BlockSpec((EB, N), lambda m: (m, 0))
    return pl.pallas_call(
        _msgs_body,
        grid=(E // EB,),
        in_specs=[
            pl.BlockSpec((EB, N, N), lambda m: (m, 0, 0)),
            pl.BlockSpec((N, N), lambda m: (0, 0)),
            eb, eb, eb, eb, eb, eb,
        ],
        out_specs=[eb, eb, eb, eb,
                   pl.BlockSpec((EB, N, 8), lambda m: (m, 0, 0))],
        out_shape=[
            jax.ShapeDtypeStruct((E, N), F32),
            jax.ShapeDtypeStruct((E, N), F32),
            jax.ShapeDtypeStruct((E, N), F32),
            jax.ShapeDtypeStruct((E, N), F32),
            jax.ShapeDtypeStruct((E, N, 8), F32),
        ],
    )(bij3, bi2, msgi2, msgj2, osrc, odst, ds, dd)


def _mm_body(kinds, *refs):
    nd = kinds.count("d")
    ng = kinds.count("g")
    it = iter(range(len(refs)))
    a_ref = refs[next(it)]
    d_refs = [refs[next(it)] for _ in range(nd)]
    o_refs = [refs[next(it)] for _ in range(ng)]
    t_refs = [refs[next(it)] for _ in range(ng)]
    outs = [refs[next(it)] for _ in range(nd + ng)]
    a = a_ref[...]
    di = gi = 0
    for gidx, kind in enumerate(kinds):
        if kind == "d":
            x_ref = d_refs[di]
            di += 1
            for j in range(NS):
                outs[gidx][:, j, :] = _bdot(a, x_ref[:, j, :])
        else:
            s = _dot(a, o_refs[gi][...])
            tref = t_refs[gi]
            gi += 1
            for j in range(NS):
                outs[gidx][:, j, :] = _bdot(s, tref[:, j, :])


def _mm_call(adj1, groups):
    B = adj1.shape[0]
    bm = min(EB, B)
    gm = B // bm
    d_arrays, o_arrays, t_arrays, kinds = [], [], [], []
    for g in groups:
        kinds.append(g[0])
        if g[0] == "d":
            d_arrays.append(g[1])
        else:
            o_arrays.append(g[1][0])
            t_arrays.append(g[1][1])
    kinds = tuple(kinds)

    in_specs = [pl.BlockSpec((bm, B), lambda m: (m, 0))]
    for arr in d_arrays + o_arrays + t_arrays:
        in_specs.append(
            pl.BlockSpec(arr.shape, lambda m, _n=arr.ndim: (0,) * _n))
    out_specs = [pl.BlockSpec((bm, NS, 128), lambda m: (m, 0, 0))
                 for _ in kinds]
    out_shape = [jax.ShapeDtypeStruct((B, NS, 128), F32) for _ in kinds]
    return pl.pallas_call(
        functools.partial(_mm_body, kinds),
        grid=(gm,),
        in_specs=in_specs,
        out_specs=out_specs,
        out_shape=out_shape,
    )(adj1, *d_arrays, *o_arrays, *t_arrays)


def _conv_body(nparts, epi, *refs):
    it = iter(range(len(refs)))
    x_refs = [refs[next(it)] for _ in range(nparts)]
    w_refs = [(refs[next(it)], refs[next(it)], refs[next(it)])
              for _ in range(nparts)]
    b_ref = refs[next(it)]
    x_extra = [refs[next(it)] for _ in range(1 if epi == "gates" else 0)]
    out_ref = refs[next(it)]
    eb = x_refs[0].shape[0]
    rb = eb * NS
    g = None
    for x_ref, (w0, w1, w2) in zip(x_refs, w_refs):
        x = x_ref[...].reshape(rb, 128)
        ri = jax.lax.broadcasted_iota(jnp.int32, (rb, 128), 0) % NS
        li = jax.lax.broadcasted_iota(jnp.int32, (rb, 128), 1)
        zrow = jnp.zeros((1, 32), F32)
        carry_m = jnp.concatenate([zrow, x[:-1, 96:]], axis=0)
        xm = jnp.concatenate([carry_m, x[:, :96]], axis=1)
        xm = jnp.where((ri == 0) & (li < 32), 0.0, xm)
        carry_p = jnp.concatenate([x[1:, :32], zrow], axis=0)
        xp = jnp.concatenate([x[:, 32:], carry_p], axis=1)
        xp = jnp.where((ri == NS - 1) & (li >= 96), 0.0, xp)
        t = _bdot(xm, w0[...]) + _bdot(x, w1[...]) + _bdot(xp, w2[...])
        g = t if g is None else g + t
    g = g + b_ref[...]
    if epi == "relu":
        out_ref[...] = jax.nn.relu(g).reshape(eb, NS, 128)
    elif epi == "gates":
        cm = x_extra[0][...].reshape(rb, 128)
        hs = []
        for s in range(4):
            gs = g[:, s * 128:(s + 1) * 128]
            gi = jax.nn.sigmoid(gs[:, :H])
            gf = jax.nn.sigmoid(gs[:, H:2 * H])
            gg = jnp.tanh(gs[:, 2 * H:3 * H])
            go = jax.nn.sigmoid(gs[:, 3 * H:])
            c_new = gf * cm[:, s * H:(s + 1) * H] + gi * gg
            hs.append(go * jnp.tanh(c_new))
        out_ref[...] = jnp.concatenate(hs, axis=1).reshape(eb, NS, 128)
    else:
        out_ref[...] = g.reshape(eb, NS, 128)


def _conv_call(parts, ws, b, epi, extra):
    B = parts[0].shape[0]
    eb = min(B, CB)
    flat_ws = [w for trio in ws for w in trio]
    in_specs = []
    for arr in parts:
        in_specs.append(pl.BlockSpec((eb, NS, 128), lambda r: (r, 0, 0)))
    for arr in flat_ws:
        in_specs.append(
            pl.BlockSpec(arr.shape, lambda r: (0, 0)))
    in_specs.append(pl.BlockSpec(b.shape, lambda r: (0, 0)))
    for arr in extra:
        in_specs.append(pl.BlockSpec((eb, NS, 128), lambda r: (r, 0, 0)))
    return pl.pallas_call(
        functools.partial(_conv_body, len(parts), epi),
        grid=(B // eb,),
        in_specs=in_specs,
        out_specs=pl.BlockSpec((eb, NS, 128), lambda r: (r, 0, 0)),
        out_shape=jax.ShapeDtypeStruct((B, NS, 128), F32),
    )(*parts, *flat_ws, b, *extra)


def _node1a_body(cmi_ref, cmj_ref, p1_ref, p2_ref, mi_ref, mj_ref,
                 ost_ref, odt_ref, bi_ref, v_ref, out_ref):
    ost = ost_ref[...]
    odt = odt_ref[...]
    ncmsgi = _dot(ost, cmi_ref[...])
    ncmsgj = _dot(odt, cmj_ref[...])
    nnmsgi = _dot(ost, mi_ref[...])
    nnmsgj = _dot(odt, mj_ref[...])
    f1 = _dot(ost, p1_ref[...])
    f2 = _dot(odt, p2_ref[...])
    bi2 = bi_ref[...]
    out_ref[...] = jnp.stack(
        [bi2, bi2 + v_ref[...], nnmsgi, nnmsgj, ncmsgi, ncmsgj, f1, f2],
        axis=-1)


def _node2a_body(ost_ref, odt_ref, ef_ref, o1_ref, o2_ref):
    ost = ost_ref[...]
    odt = odt_ref[...]
    for j in range(NS):
        efj = ef_ref[:, j, :]
        o1_ref[:, j, :] = _bdot(ost, efj)
        o2_ref[:, j, :] = _bdot(odt, efj)


def _final_body(ef3_ref, bij_ref, cmi_ref, cmj_ref, mi_ref, mj_ref, out_ref):
    ef3 = ef3_ref[...]
    lane = jax.lax.broadcasted_iota(jnp.int32, ef3.shape, 2)
    efin0 = jnp.sum(jnp.where(lane == 0, ef3, 0.0), axis=-1)
    efin1 = jnp.sum(jnp.where(lane == 1, ef3, 0.0), axis=-1)
    nmsgi = efin0 + 0.5 * cmi_ref[...] - mi_ref[...]
    nmsgj = efin1 + 0.5 * cmj_ref[...] - mj_ref[...]
    out_ref[...] = bij_ref[...] - nmsgi[:, :, None] - nmsgj[:, None, :]


def _wprep(W, b, cin=None, cout=None):
    w0, w1, w2 = W[:, :, 0].T, W[:, :, 1].T, W[:, :, 2].T
    bb = b.reshape(1, -1)
    if cin is not None and w0.shape[0] < cin:
        z = jnp.zeros((cin - w0.shape[0], w0.shape[1]), F32)
        w0, w1, w2 = (jnp.concatenate([w, z], axis=0) for w in (w0, w1, w2))
    if cout is not None and w0.shape[1] < cout:
        z = jnp.zeros((w0.shape[0], cout - w0.shape[1]), F32)
        w0, w1, w2 = (jnp.concatenate([w, z], axis=1) for w in (w0, w1, w2))
        bb = jnp.concatenate(
            [bb, jnp.zeros((1, cout - bb.shape[1]), F32)], axis=1)
    eye4 = jnp.eye(4, dtype=F32)
    return ([jnp.kron(eye4, w) for w in (w0, w1, w2)],
            jnp.tile(bb, (1, 4)))


def _wsplit(wtrip, nparts):
    ws_full, b = wtrip
    ctot = ws_full[0].shape[0] // 4
    out = []
    for i in range(nparts):
        trio = []
        for w in ws_full:
            rows = [w[s * ctot + 32 * i: s * ctot + 32 * i + 32]
                    for s in range(4)]
            trio.append(jnp.concatenate(rows, axis=0))
        out.append(tuple(trio))
    return out, b


def _layer(adj1, groups, wtrip, epi, extra):
    ys = _mm_call(adj1, groups)
    ws, b = _wsplit(wtrip, len(ys))
    return _conv_call(ys, ws, b, epi, extra)


def _pack(x3):
    B, n, c = x3.shape
    if c < 32:
        x3 = jnp.pad(x3, ((0, 0), (0, 0), (0, 32 - c)))
    return x3.reshape(B, NS, 128)


def kernel(bi, bij, msgi, msgj, v, nmems, emems, neadj, eeadj, edge_index,
           decoding, params):
    src = edge_index[0]
    dst = edge_index[1]
    bi2 = bi.reshape(N, N)
    bij3 = bij.reshape(E, N, N)
    msgi2 = msgi.reshape(E, N)
    msgj2 = msgj.reshape(E, N)
    v2 = v.reshape(1, N)
    iota = jnp.arange(N, dtype=jnp.int32)
    osrc = (src[:, None] == iota[None, :]).astype(F32)
    odst = (dst[:, None] == iota[None, :]).astype(F32)
    ost = (iota[:, None] == src[None, :]).astype(F32)
    odt = (iota[:, None] == dst[None, :]).astype(F32)
    ds = (decoding[src][:, None] == iota[None, :]).astype(F32)
    dd = (decoding[dst][:, None] == iota[None, :]).astype(F32)
    nmemt = jnp.swapaxes(nmems, 2, 3)
    ememt = jnp.swapaxes(emems, 2, 3)
    em0c = _pack(ememt[0, :, :, :H])
    em0h = _pack(ememt[0, :, :, H:])
    em1c = _pack(ememt[1, :, :, :H])
    em1h = _pack(ememt[1, :, :, H:])
    nm0c = _pack(nmemt[0, :, :, :H])
    nm0h = _pack(nmemt[0, :, :, H:])
    nm1c = _pack(nmemt[1, :, :, :H])
    nm1h = _pack(nmemt[1, :, :, H:])

    cmsgi, cmsgj, p1, p2, ef0 = _msgs_call(
        bij3, bi2, msgi2, msgj2, osrc, odst, ds, dd)

    nf3 = pl.pallas_call(
        _node1a_body,
        grid=(1,),
        in_specs=[pl.BlockSpec(x.shape, lambda i, _n=x.ndim: (0,) * _n)
                  for x in (cmsgi, cmsgj, p1, p2, msgi2, msgj2, ost, odt,
                            bi2, v2)],
        out_specs=pl.BlockSpec((N, N, 8), lambda i: (0, 0, 0)),
        out_shape=jax.ShapeDtypeStruct((N, N, 8), F32),
    )(cmsgi, cmsgj, p1, p2, msgi2, msgj2, ost, odt, bi2, v2)

    ee1 = eeadj + jnp.eye(E, dtype=F32)
    ne1 = neadj + jnp.eye(N, dtype=F32)

    (fm0W, fm0b), (fm1W, fm1b) = params["fm"]
    nh = _layer(ne1, [("d", _pack(nf3))], _wprep(fm0W, fm0b, cin=32),
                "relu", [])
    nfeat = _layer(ne1, [("d", nh)], _wprep(fm1W, fm1b), "relu", [])

    (em0W, em0b), (em1W, em1b) = params["efm"]
    e1 = _layer(ee1, [("d", _pack(ef0))], _wprep(em0W, em0b, cin=32),
                "relu", [])
    efeat = _layer(ee1, [("d", e1)], _wprep(em1W, em1b), "relu", [])

    esrc, edst = pl.pallas_call(
        _node2a_body,
        grid=(1,),
        in_specs=[pl.BlockSpec(x.shape, lambda i, _n=x.ndim: (0,) * _n)
                  for x in (ost, odt, efeat)],
        out_specs=[pl.BlockSpec((N, NS, 128), lambda i: (0, 0, 0))] * 2,
        out_shape=[jax.ShapeDtypeStruct((N, NS, 128), F32)] * 2,
    )(ost, odt, efeat)
    (ln0W, ln0b) = params["lstm"][0]
    (ln1W, ln1b) = params["lstm"][1]
    nh0 = _layer(ne1, [("d", nfeat), ("d", esrc), ("d", edst),
                       ("d", nm0h)],
                 _wprep(ln0W, ln0b), "gates", [nm0c])
    nh1 = _layer(ne1, [("d", nh0), ("d", nm1h)],
                 _wprep(ln1W, ln1b), "gates", [nm1c])

    (le0W, le0b) = params["elstm"][0]
    (le1W, le1b) = params["elstm"][1]
    h0 = _layer(ee1, [("d", efeat), ("g", (osrc, nfeat)),
                      ("g", (odst, nfeat)), ("d", em0h)],
                _wprep(le0W, le0b), "gates", [em0c])
    h1 = _layer(ee1, [("d", h0), ("d", em1h)],
                _wprep(le1W, le1b), "gates", [em1c])

    (p0W, p0b), (p1W, p1b) = params["epost"]
    q = _layer(ee1, [("d", h1), ("g", (osrc, nh1)), ("g", (odst, nh1))],
               _wprep(p0W, p0b), "relu", [])
    efin = _layer(ee1, [("d", q)], _wprep(p1W, p1b, cout=32), "none", [])
    ef3 = efin.reshape(E, N, 32)

    bij_new = pl.pallas_call(
        _final_body,
        grid=(E // EB,),
        in_specs=[
            pl.BlockSpec((EB, N, 32), lambda m: (m, 0, 0)),
            pl.BlockSpec((EB, N, N), lambda m: (m, 0, 0)),
            pl.BlockSpec((EB, N), lambda m: (m, 0)),
            pl.BlockSpec((EB, N), lambda m: (m, 0)),
            pl.BlockSpec((EB, N), lambda m: (m, 0)),
            pl.BlockSpec((EB, N), lambda m: (m, 0)),
        ],
        out_specs=pl.BlockSpec((EB, N, N), lambda m: (m, 0, 0)),
        out_shape=jax.ShapeDtypeStruct((E, N, N), F32),
    )(ef3, bij3, cmsgi, cmsgj, msgi2, msgj2)
    return bij_new.reshape(E, 1, N, N)


# speedup vs baseline: 1.6651x; 1.6651x over previous
import functools

import jax
import jax.numpy as jnp
from jax.experimental import pallas as pl
from jax.experimental.pallas import tpu as pltpu

N = 64
E = 1024
H = 32
EPS = 20.0
EB = 128
CB = 128
NS = 16
F32 = jnp.float32


def _dot(a, b):
    return jnp.dot(a, b, preferred_element_type=F32)


def _bdot(a, b):
    return jnp.dot(a.astype(jnp.bfloat16), b.astype(jnp.bfloat16),
                   preferred_element_type=F32)


def _msgs_body(bij_ref, bi_ref, mi_ref, mj_ref, os_ref, od_ref, ds_ref,
               dd_ref, cmi_ref, cmj_ref, p1_ref, p2_ref, ef_ref):
    bijb = bij_ref[...]
    bi2 = bi_ref[...]
    ntoe1 = _dot(os_ref[...], bi2)
    ntoe2 = _dot(od_ref[...], bi2)
    t = EPS * (bijb + ntoe2[:, None, :])
    m1 = jnp.max(t, axis=2)
    cmi = (m1 + jnp.log(jnp.sum(jnp.exp(t - m1[:, :, None]), axis=2))) / EPS
    t2 = EPS * (bijb + ntoe1[:, :, None])
    m2 = jnp.max(t2, axis=1)
    cmj = (m2 + jnp.log(jnp.sum(jnp.exp(t2 - m2[:, None, :]), axis=1))) / EPS
    p1 = jnp.sum(bijb * ds_ref[...][:, :, None], axis=1)
    p2 = jnp.sum(bijb * dd_ref[...][:, None, :], axis=2)
    cmi_ref[...] = cmi
    cmj_ref[...] = cmj
    p1_ref[...] = p1
    p2_ref[...] = p2
    zero = jnp.zeros_like(cmi)
    ef_ref[...] = jnp.stack(
        [mi_ref[...], mj_ref[...], cmi, cmj, ntoe1, ntoe2] + [zero] * 2,
        axis=-1)


def _msgs_call(bij3, bi2, msgi2, msgj2, osrc, odst, ds, dd):
    eb = pl.BlockSpec((EB, N), lambda m: (m, 0))
    return pl.pallas_call(
        _msgs_body,
        grid=(E // EB,),
        in_specs=[
            pl.BlockSpec((EB, N, N), lambda m: (m, 0, 0)),
            pl.BlockSpec((N, N), lambda m: (0, 0)),
            eb, eb, eb, eb, eb, eb,
        ],
        out_specs=[eb, eb, eb, eb,
                   pl.BlockSpec((EB, N, 8), lambda m: (m, 0, 0))],
        out_shape=[
            jax.ShapeDtypeStruct((E, N), F32),
            jax.ShapeDtypeStruct((E, N), F32),
            jax.ShapeDtypeStruct((E, N), F32),
            jax.ShapeDtypeStruct((E, N), F32),
            jax.ShapeDtypeStruct((E, N, 8), F32),
        ],
    )(bij3, bi2, msgi2, msgj2, osrc, odst, ds, dd)


def _mm_body(kinds, *refs):
    nd = kinds.count("d")
    ng = kinds.count("g")
    it = iter(range(len(refs)))
    a_ref = refs[next(it)]
    d_refs = [refs[next(it)] for _ in range(nd)]
    o_refs = [refs[next(it)] for _ in range(ng)]
    t_refs = [refs[next(it)] for _ in range(ng)]
    outs = [refs[next(it)] for _ in range(nd + ng)]
    a = a_ref[...]
    di = gi = 0
    for gidx, kind in enumerate(kinds):
        if kind == "d":
            x_ref = d_refs[di]
            di += 1
            for j in range(NS):
                outs[gidx][j] = _bdot(a, x_ref[j])
        else:
            s = _dot(a, o_refs[gi][...])
            tref = t_refs[gi]
            gi += 1
            for j in range(NS):
                outs[gidx][j] = _bdot(s, tref[j])


def _mm_call(adj1, groups):
    B = adj1.shape[0]
    bm = min(EB, B)
    gm = B // bm
    d_arrays, o_arrays, t_arrays, kinds = [], [], [], []
    for g in groups:
        kinds.append(g[0])
        if g[0] == "d":
            d_arrays.append(g[1])
        else:
            o_arrays.append(g[1][0])
            t_arrays.append(g[1][1])
    kinds = tuple(kinds)

    in_specs = [pl.BlockSpec((bm, B), lambda m: (m, 0))]
    for arr in d_arrays + o_arrays + t_arrays:
        in_specs.append(
            pl.BlockSpec(arr.shape, lambda m, _n=arr.ndim: (0,) * _n))
    out_specs = [pl.BlockSpec((NS, bm, 128), lambda m: (0, m, 0))
                 for _ in kinds]
    out_shape = [jax.ShapeDtypeStruct((NS, B, 128), F32) for _ in kinds]
    return pl.pallas_call(
        functools.partial(_mm_body, kinds),
        grid=(gm,),
        in_specs=in_specs,
        out_specs=out_specs,
        out_shape=out_shape,
    )(adj1, *d_arrays, *o_arrays, *t_arrays)


def _conv_body(nparts, epi, *refs):
    it = iter(range(len(refs)))
    x_refs = [refs[next(it)] for _ in range(nparts)]
    w_refs = [(refs[next(it)], refs[next(it)], refs[next(it)])
              for _ in range(nparts)]
    b_ref = refs[next(it)]
    x_extra = [refs[next(it)] for _ in range(1 if epi == "gates" else 0)]
    out_ref = refs[next(it)]
    eb = x_refs[0].shape[1]
    rb = eb * NS
    g = None
    for x_ref, (w0, w1, w2) in zip(x_refs, w_refs):
        x = x_ref[...].reshape(rb, 128)
        ri = jax.lax.broadcasted_iota(jnp.int32, (rb, 128), 0)
        li = jax.lax.broadcasted_iota(jnp.int32, (rb, 128), 1)
        zrow = jnp.zeros((eb, 32), F32)
        carry_m = jnp.concatenate([zrow, x[:-eb, 96:]], axis=0)
        xm = jnp.concatenate([carry_m, x[:, :96]], axis=1)
        xm = jnp.where((ri < eb) & (li < 32), 0.0, xm)
        carry_p = jnp.concatenate([x[eb:, :32], zrow], axis=0)
        xp = jnp.concatenate([x[:, 32:], carry_p], axis=1)
        xp = jnp.where((ri >= rb - eb) & (li >= 96), 0.0, xp)
        t = _bdot(xm, w0[...]) + _bdot(x, w1[...]) + _bdot(xp, w2[...])
        g = t if g is None else g + t
    g = g + b_ref[...]
    if epi == "relu":
        out_ref[...] = jax.nn.relu(g).reshape(NS, eb, 128)
    elif epi == "gates":
        cm = x_extra[0][...].reshape(rb, 128)
        hs = []
        for s in range(4):
            gs = g[:, s * 128:(s + 1) * 128]
            gi = jax.nn.sigmoid(gs[:, :H])
            gf = jax.nn.sigmoid(gs[:, H:2 * H])
            gg = jnp.tanh(gs[:, 2 * H:3 * H])
            go = jax.nn.sigmoid(gs[:, 3 * H:])
            c_new = gf * cm[:, s * H:(s + 1) * H] + gi * gg
            hs.append(go * jnp.tanh(c_new))
        out_ref[...] = jnp.concatenate(hs, axis=1).reshape(NS, eb, 128)
    else:
        out_ref[...] = g.reshape(NS, eb, 128)


def _conv_call(parts, ws, b, epi, extra):
    B = parts[0].shape[1]
    eb = min(B, CB)
    flat_ws = [w for trio in ws for w in trio]
    in_specs = []
    for arr in parts:
        in_specs.append(pl.BlockSpec((NS, eb, 128), lambda r: (0, r, 0)))
    for arr in flat_ws:
        in_specs.append(
            pl.BlockSpec(arr.shape, lambda r: (0, 0)))
    in_specs.append(pl.BlockSpec(b.shape, lambda r: (0, 0)))
    for arr in extra:
        in_specs.append(pl.BlockSpec((NS, eb, 128), lambda r: (0, r, 0)))
    return pl.pallas_call(
        functools.partial(_conv_body, len(parts), epi),
        grid=(B // eb,),
        in_specs=in_specs,
        out_specs=pl.BlockSpec((NS, eb, 128), lambda r: (0, r, 0)),
        out_shape=jax.ShapeDtypeStruct((NS, B, 128), F32),
    )(*parts, *flat_ws, b, *extra)


def _node1a_body(cmi_ref, cmj_ref, p1_ref, p2_ref, mi_ref, mj_ref,
                 ost_ref, odt_ref, bi_ref, v_ref, out_ref):
    ost = ost_ref[...]
    odt = odt_ref[...]
    ncmsgi = _dot(ost, cmi_ref[...])
    ncmsgj = _dot(odt, cmj_ref[...])
    nnmsgi = _dot(ost, mi_ref[...])
    nnmsgj = _dot(odt, mj_ref[...])
    f1 = _dot(ost, p1_ref[...])
    f2 = _dot(odt, p2_ref[...])
    bi2 = bi_ref[...]
    out_ref[...] = jnp.stack(
        [bi2, bi2 + v_ref[...], nnmsgi, nnmsgj, ncmsgi, ncmsgj, f1, f2],
        axis=-1)


def _node2a_body(ost_ref, odt_ref, ef_ref, o1_ref, o2_ref):
    ost = ost_ref[...]
    odt = odt_ref[...]
    for j in range(NS):
        efj = ef_ref[j]
        o1_ref[j] = _bdot(ost, efj)
        o2_ref[j] = _bdot(odt, efj)


def _final_body(ef3_ref, bij_ref, cmi_ref, cmj_ref, mi_ref, mj_ref, out_ref):
    ef3 = ef3_ref[...]
    lane = jax.lax.broadcasted_iota(jnp.int32, ef3.shape, 2)
    efin0 = jnp.sum(jnp.where(lane == 0, ef3, 0.0), axis=-1)
    efin1 = jnp.sum(jnp.where(lane == 1, ef3, 0.0), axis=-1)
    nmsgi = efin0 + 0.5 * cmi_ref[...] - mi_ref[...]
    nmsgj = efin1 + 0.5 * cmj_ref[...] - mj_ref[...]
    out_ref[...] = bij_ref[...] - nmsgi[:, :, None] - nmsgj[:, None, :]


def _wprep(W, b, cin=None, cout=None):
    w0, w1, w2 = W[:, :, 0].T, W[:, :, 1].T, W[:, :, 2].T
    bb = b.reshape(1, -1)
    if cin is not None and w0.shape[0] < cin:
        z = jnp.zeros((cin - w0.shape[0], w0.shape[1]), F32)
        w0, w1, w2 = (jnp.concatenate([w, z], axis=0) for w in (w0, w1, w2))
    if cout is not None and w0.shape[1] < cout:
        z = jnp.zeros((w0.shape[0], cout - w0.shape[1]), F32)
        w0, w1, w2 = (jnp.concatenate([w, z], axis=1) for w in (w0, w1, w2))
        bb = jnp.concatenate(
            [bb, jnp.zeros((1, cout - bb.shape[1]), F32)], axis=1)
    eye4 = jnp.eye(4, dtype=F32)
    return ([jnp.kron(eye4, w) for w in (w0, w1, w2)],
            jnp.tile(bb, (1, 4)))


def _wsplit(wtrip, nparts):
    ws_full, b = wtrip
    ctot = ws_full[0].shape[0] // 4
    out = []
    for i in range(nparts):
        trio = []
        for w in ws_full:
            rows = [w[s * ctot + 32 * i: s * ctot + 32 * i + 32]
                    for s in range(4)]
            trio.append(jnp.concatenate(rows, axis=0))
        out.append(tuple(trio))
    return out, b


def _layer(adj1, groups, wtrip, epi, extra):
    ys = _mm_call(adj1, groups)
    ws, b = _wsplit(wtrip, len(ys))
    return _conv_call(ys, ws, b, epi, extra)


def _pack(x3):
    B, n, c = x3.shape
    if c < 32:
        x3 = jnp.pad(x3, ((0, 0), (0, 0), (0, 32 - c)))
    return jnp.swapaxes(x3.reshape(B, NS, 128), 0, 1)


def kernel(bi, bij, msgi, msgj, v, nmems, emems, neadj, eeadj, edge_index,
           decoding, params):
    src = edge_index[0]
    dst = edge_index[1]
    bi2 = bi.reshape(N, N)
    bij3 = bij.reshape(E, N, N)
    msgi2 = msgi.reshape(E, N)
    msgj2 = msgj.reshape(E, N)
    v2 = v.reshape(1, N)
    iota = jnp.arange(N, dtype=jnp.int32)
    osrc = (src[:, None] == iota[None, :]).astype(F32)
    odst = (dst[:, None] == iota[None, :]).astype(F32)
    ost = (iota[:, None] == src[None, :]).astype(F32)
    odt = (iota[:, None] == dst[None, :]).astype(F32)
    ds = (decoding[src][:, None] == iota[None, :]).astype(F32)
    dd = (decoding[dst][:, None] == iota[None, :]).astype(F32)
    nmemt = jnp.swapaxes(nmems, 2, 3)
    ememt = jnp.swapaxes(emems, 2, 3)
    em0c = _pack(ememt[0, :, :, :H])
    em0h = _pack(ememt[0, :, :, H:])
    em1c = _pack(ememt[1, :, :, :H])
    em1h = _pack(ememt[1, :, :, H:])
    nm0c = _pack(nmemt[0, :, :, :H])
    nm0h = _pack(nmemt[0, :, :, H:])
    nm1c = _pack(nmemt[1, :, :, :H])
    nm1h = _pack(nmemt[1, :, :, H:])

    cmsgi, cmsgj, p1, p2, ef0 = _msgs_call(
        bij3, bi2, msgi2, msgj2, osrc, odst, ds, dd)

    nf3 = pl.pallas_call(
        _node1a_body,
        grid=(1,),
        in_specs=[pl.BlockSpec(x.shape, lambda i, _n=x.ndim: (0,) * _n)
                  for x in (cmsgi, cmsgj, p1, p2, msgi2, msgj2, ost, odt,
                            bi2, v2)],
        out_specs=pl.BlockSpec((N, N, 8), lambda i: (0, 0, 0)),
        out_shape=jax.ShapeDtypeStruct((N, N, 8), F32),
    )(cmsgi, cmsgj, p1, p2, msgi2, msgj2, ost, odt, bi2, v2)

    ee1 = eeadj + jnp.eye(E, dtype=F32)
    ne1 = neadj + jnp.eye(N, dtype=F32)

    (fm0W, fm0b), (fm1W, fm1b) = params["fm"]
    nh = _layer(ne1, [("d", _pack(nf3))], _wprep(fm0W, fm0b, cin=32),
                "relu", [])
    nfeat = _layer(ne1, [("d", nh)], _wprep(fm1W, fm1b), "relu", [])

    (em0W, em0b), (em1W, em1b) = params["efm"]
    e1 = _layer(ee1, [("d", _pack(ef0))], _wprep(em0W, em0b, cin=32),
                "relu", [])
    efeat = _layer(ee1, [("d", e1)], _wprep(em1W, em1b), "relu", [])

    esrc, edst = pl.pallas_call(
        _node2a_body,
        grid=(1,),
        in_specs=[pl.BlockSpec(x.shape, lambda i, _n=x.ndim: (0,) * _n)
                  for x in (ost, odt, efeat)],
        out_specs=[pl.BlockSpec((NS, N, 128), lambda i: (0, 0, 0))] * 2,
        out_shape=[jax.ShapeDtypeStruct((NS, N, 128), F32)] * 2,
    )(ost, odt, efeat)
    (ln0W, ln0b) = params["lstm"][0]
    (ln1W, ln1b) = params["lstm"][1]
    nh0 = _layer(ne1, [("d", nfeat), ("d", esrc), ("d", edst),
                       ("d", nm0h)],
                 _wprep(ln0W, ln0b), "gates", [nm0c])
    nh1 = _layer(ne1, [("d", nh0), ("d", nm1h)],
                 _wprep(ln1W, ln1b), "gates", [nm1c])

    (le0W, le0b) = params["elstm"][0]
    (le1W, le1b) = params["elstm"][1]
    h0 = _layer(ee1, [("d", efeat), ("g", (osrc, nfeat)),
                      ("g", (odst, nfeat)), ("d", em0h)],
                _wprep(le0W, le0b), "gates", [em0c])
    h1 = _layer(ee1, [("d", h0), ("d", em1h)],
                _wprep(le1W, le1b), "gates", [em1c])

    (p0W, p0b), (p1W, p1b) = params["epost"]
    q = _layer(ee1, [("d", h1), ("g", (osrc, nh1)), ("g", (odst, nh1))],
               _wprep(p0W, p0b), "relu", [])
    efin = _layer(ee1, [("d", q)], _wprep(p1W, p1b, cout=32), "none", [])
    ef3 = jnp.swapaxes(efin, 0, 1).reshape(E, N, 32)

    bij_new = pl.pallas_call(
        _final_body,
        grid=(E // EB,),
        in_specs=[
            pl.BlockSpec((EB, N, 32), lambda m: (m, 0, 0)),
            pl.BlockSpec((EB, N, N), lambda m: (m, 0, 0)),
            pl.BlockSpec((EB, N), lambda m: (m, 0)),
            pl.BlockSpec((EB, N), lambda m: (m, 0)),
            pl.BlockSpec((EB, N), lambda m: (m, 0)),
            pl.BlockSpec((EB, N), lambda m: (m, 0)),
        ],
        out_specs=pl.BlockSpec((EB, N, N), lambda m: (m, 0, 0)),
        out_shape=jax.ShapeDtypeStruct((E, N, N), F32),
    )(ef3, bij3, cmsgi, cmsgj, msgi2, msgj2)
    return bij_new.reshape(E, 1, N, N)
